# baseline (speedup 1.0000x reference)
import jax
import jax.numpy as jnp
import numpy as np
from jax import lax
from jax.experimental import pallas as pl
from jax.experimental.pallas import tpu as pltpu
from jax.experimental.pallas import tpu_sc as plsc

VOCAB = 100000
D = 768
BATCH = 4
SEQ = 4096
LANES = 16
D_VECS = D // LANES

NC = 2
NS = 16
NW = NC * NS

POS_PER_W = SEQ // NW
CHUNK = 32
N_CHUNKS = POS_PER_W // CHUNK
ROUNDS = N_CHUNKS * BATCH
NB = 3
NPE = 2
LOOKAHEAD = 1


def _pe_table():
    pos = np.arange(SEQ, dtype=np.float32)[:, None]
    i = np.arange(0, D, 2, dtype=np.float32)
    div = np.power(np.float32(10000.0), i / np.float32(D))
    pe = np.zeros((SEQ, D), dtype=np.float32)
    pe[:, 0::2] = np.sin(pos / div, dtype=np.float32)
    pe[:, 1::2] = np.cos(pos / div, dtype=np.float32)
    return jnp.asarray(pe.reshape(SEQ * D))


def _sc_body(x_hbm, pe_hbm, tab_hbm, out_hbm, idx_v, rows, pe_v,
             pe_sem, g_sem, st_sem):
    wid = lax.axis_index("s") * NC + lax.axis_index("c")
    pos0 = wid * POS_PER_W

    for b in range(BATCH):
        pltpu.sync_copy(x_hbm.at[pl.ds(b * SEQ + pos0, POS_PER_W)], idx_v.at[b])

    def cb(r):
        return r // BATCH, r % BATCH

    def issue_pe(c):
        return pltpu.async_copy(
            pe_hbm.at[pl.ds((pos0 + c * CHUNK) * D, CHUNK * D)], pe_v[c % NPE],
            pe_sem[c % NPE])

    def issue_g(r):
        c, b = cb(r)
        return pltpu.async_copy(
            tab_hbm.at[idx_v.at[b, pl.ds(c * CHUNK, CHUNK)]],
            rows[r % NB], g_sem[r % NB])

    def issue_st(r):
        c, b = cb(r)
        dst = b * SEQ + pos0 + c * CHUNK
        return pltpu.async_copy(
            rows[r % NB], out_hbm.at[pl.ds(dst, CHUNK)], st_sem[r % NB])

    def add_pe(r):
        c, _ = cb(r)
        rbuf, pbuf = rows[r % NB], pe_v[c % NPE]

        def body(i, _):
            for j in range(D_VECS):
                sl = pl.ds(j * LANES, LANES)
                plsc.addupdate(rbuf.at[i, sl], pbuf[pl.ds(i * D + j * LANES, LANES)])
            return 0

        lax.fori_loop(0, CHUNK, body, 0)

    d_pe, d_g, d_st = {}, {}, {}
    d_pe[0] = issue_pe(0)
    d_pe[1] = issue_pe(1)
    for r in range(LOOKAHEAD):
        d_g[r] = issue_g(r)
    for r in range(ROUNDS):
        c, b = cb(r)
        if r + LOOKAHEAD < ROUNDS:
            if r - (NB - LOOKAHEAD) >= 0:
                d_st[r - (NB - LOOKAHEAD)].wait()
            d_g[r + LOOKAHEAD] = issue_g(r + LOOKAHEAD)
        d_g[r].wait()
        if b == 0:
            d_pe[c].wait()
        add_pe(r)
        d_st[r] = issue_st(r)
        if b == BATCH - 1 and c + 2 < N_CHUNKS:
            d_pe[c + 2] = issue_pe(c + 2)
    for r in range(ROUNDS - NB, ROUNDS):
        if r in d_st:
            d_st[r].wait()


@jax.jit
def kernel(x, tok_table):
    pe = _pe_table()
    x_flat = x.reshape(BATCH * SEQ).astype(jnp.int32)

    mesh = plsc.VectorSubcoreMesh(core_axis_name="c", subcore_axis_name="s")
    run = pl.kernel(
        _sc_body,
        out_type=jax.ShapeDtypeStruct((BATCH * SEQ, D), jnp.float32),
        mesh=mesh,
        scratch_types=[
            pltpu.VMEM((BATCH, POS_PER_W), jnp.int32),
            [pltpu.VMEM((CHUNK, D), jnp.float32) for _ in range(NB)],
            [pltpu.VMEM((CHUNK * D,), jnp.float32) for _ in range(NPE)],
            [pltpu.SemaphoreType.DMA for _ in range(NPE)],
            [pltpu.SemaphoreType.DMA for _ in range(NB)],
            [pltpu.SemaphoreType.DMA for _ in range(NB)],
        ],
    )
    out = run(x_flat, pe, tok_table)
    return out.reshape(BATCH, SEQ, D)


# speedup vs baseline: 1.5669x; 1.5669x over previous
import jax
import jax.numpy as jnp
import numpy as np
from jax import lax
from jax.experimental import pallas as pl
from jax.experimental.pallas import tpu as pltpu
from jax.experimental.pallas import tpu_sc as plsc

VOCAB = 100000
D = 768
BATCH = 4
SEQ = 4096
LANES = 16
D_VECS = D // LANES

NC = 2
NS = 16
NW = NC * NS

POS_PER_W = SEQ // NW
CHUNK = 32
N_CHUNKS = POS_PER_W // CHUNK
ROUNDS = N_CHUNKS * BATCH
NB = 3
NPE = 2
LOOKAHEAD = 1


def _pe_table():
    pos = np.arange(SEQ, dtype=np.float32)[:, None]
    i = np.arange(0, D, 2, dtype=np.float32)
    div = np.power(np.float32(10000.0), i / np.float32(D))
    pe = np.zeros((SEQ, D), dtype=np.float32)
    pe[:, 0::2] = np.sin(pos / div, dtype=np.float32)
    pe[:, 1::2] = np.cos(pos / div, dtype=np.float32)
    return jnp.asarray(pe)


def _sc_body(x_hbm, pe_hbm, tab_hbm, out_hbm, idx_v, rows, pe_v,
             pe_sem, g_sem, st_sem):
    wid = lax.axis_index("s") * NC + lax.axis_index("c")
    pos0 = wid * POS_PER_W

    for b in range(BATCH):
        pltpu.sync_copy(x_hbm.at[pl.ds(b * SEQ + pos0, POS_PER_W)], idx_v.at[b])

    def cb(r):
        return r // BATCH, r % BATCH

    def issue_pe(c):
        return pltpu.async_copy(
            pe_hbm.at[pl.ds(pos0 + c * CHUNK, CHUNK)], pe_v[c % NPE],
            pe_sem[c % NPE])

    def issue_g(r):
        c, b = cb(r)
        return pltpu.async_copy(
            tab_hbm.at[idx_v.at[b, pl.ds(c * CHUNK, CHUNK)]],
            rows[r % NB], g_sem[r % NB])

    def issue_st(r):
        c, b = cb(r)
        dst = b * SEQ + pos0 + c * CHUNK
        return pltpu.async_copy(
            rows[r % NB], out_hbm.at[pl.ds(dst, CHUNK)], st_sem[r % NB])

    def add_pe(r):
        c, _ = cb(r)
        rbuf, pbuf = rows[r % NB], pe_v[c % NPE]

        def body(i, _):
            for j in range(D_VECS):
                sl = pl.ds(j * LANES, LANES)
                plsc.addupdate(rbuf.at[i, sl], pbuf[i, sl])
            return 0

        lax.fori_loop(0, CHUNK, body, 0)

    d_pe, d_g, d_st = {}, {}, {}
    d_pe[0] = issue_pe(0)
    d_pe[1] = issue_pe(1)
    for r in range(LOOKAHEAD):
        d_g[r] = issue_g(r)
    for r in range(ROUNDS):
        c, b = cb(r)
        if r + LOOKAHEAD < ROUNDS:
            if r - (NB - LOOKAHEAD) >= 0:
                d_st[r - (NB - LOOKAHEAD)].wait()
            d_g[r + LOOKAHEAD] = issue_g(r + LOOKAHEAD)
        d_g[r].wait()
        if b == 0:
            d_pe[c].wait()
        add_pe(r)
        d_st[r] = issue_st(r)
        if b == BATCH - 1 and c + 2 < N_CHUNKS:
            d_pe[c + 2] = issue_pe(c + 2)
    for r in range(ROUNDS - NB, ROUNDS):
        if r in d_st:
            d_st[r].wait()


@jax.jit
def kernel(x, tok_table):
    pe = _pe_table()
    x_flat = x.reshape(BATCH * SEQ).astype(jnp.int32)

    mesh = plsc.VectorSubcoreMesh(core_axis_name="c", subcore_axis_name="s")
    run = pl.kernel(
        _sc_body,
        out_type=jax.ShapeDtypeStruct((BATCH * SEQ, D), jnp.float32),
        mesh=mesh,
        scratch_types=[
            pltpu.VMEM((BATCH, POS_PER_W), jnp.int32),
            [pltpu.VMEM((CHUNK, D), jnp.float32) for _ in range(NB)],
            [pltpu.VMEM((CHUNK, D), jnp.float32) for _ in range(NPE)],
            [pltpu.SemaphoreType.DMA for _ in range(NPE)],
            [pltpu.SemaphoreType.DMA for _ in range(NB)],
            [pltpu.SemaphoreType.DMA for _ in range(NB)],
        ],
    )
    out = run(x_flat, pe, tok_table)
    return out.reshape(BATCH, SEQ, D)


# speedup vs baseline: 1.5998x; 1.0210x over previous
import jax
import jax.numpy as jnp
import numpy as np
from jax import lax
from jax.experimental import pallas as pl
from jax.experimental.pallas import tpu as pltpu
from jax.experimental.pallas import tpu_sc as plsc

VOCAB = 100000
D = 768
BATCH = 4
SEQ = 4096
LANES = 16
D_VECS = D // LANES

NC = 2
NS = 16
NW = NC * NS

POS_PER_W = SEQ // NW
CHUNK = 32
N_CHUNKS = POS_PER_W // CHUNK
ROUNDS = N_CHUNKS * BATCH
NB = 3
NPE = 2
LOOKAHEAD = 1


def _pe_table():
    pos = np.arange(SEQ, dtype=np.float32)[:, None]
    i = np.arange(0, D, 2, dtype=np.float32)
    div = np.power(np.float32(10000.0), i / np.float32(D))
    pe = np.zeros((SEQ, D), dtype=np.float32)
    pe[:, 0::2] = np.sin(pos / div, dtype=np.float32)
    pe[:, 1::2] = np.cos(pos / div, dtype=np.float32)
    return jnp.asarray(pe)


def _sc_body(x_hbm, pe_hbm, tab_hbm, out_hbm, idx_v, rows, pe_v,
             pe_sem, g_sem, st_sem):
    wid = lax.axis_index("s") * NC + lax.axis_index("c")
    pos0 = wid * POS_PER_W

    for b in range(BATCH):
        pltpu.sync_copy(x_hbm.at[pl.ds(b * SEQ + pos0, POS_PER_W)], idx_v.at[b])

    def cb(r):
        return r // BATCH, r % BATCH

    def issue_pe(c):
        return pltpu.async_copy(
            pe_hbm.at[pl.ds(pos0 + c * CHUNK, CHUNK)], pe_v[c % NPE],
            pe_sem[c % NPE])

    def issue_g(r):
        c, b = cb(r)
        return pltpu.async_copy(
            tab_hbm.at[idx_v.at[b, pl.ds(c * CHUNK, CHUNK)]],
            rows[r % NB], g_sem[r % NB])

    def issue_st(r):
        c, b = cb(r)
        dst = b * SEQ + pos0 + c * CHUNK
        return pltpu.async_copy(
            rows[r % NB], out_hbm.at[pl.ds(dst, CHUNK)], st_sem[r % NB])

    def add_pe(r):
        c, _ = cb(r)
        rbuf, pbuf = rows[r % NB], pe_v[c % NPE]

        def body(i2, _):
            for di in range(2):
                i = i2 * 2 + di
                for j in range(D_VECS):
                    sl = pl.ds(j * LANES, LANES)
                    plsc.addupdate(rbuf.at[i, sl], pbuf[i, sl])
            return 0

        lax.fori_loop(0, CHUNK // 2, body, 0)

    d_pe, d_g, d_st = {}, {}, {}
    d_pe[0] = issue_pe(0)
    d_pe[1] = issue_pe(1)
    for r in range(LOOKAHEAD):
        d_g[r] = issue_g(r)
    for r in range(ROUNDS):
        c, b = cb(r)
        if r + LOOKAHEAD < ROUNDS:
            if r - (NB - LOOKAHEAD) >= 0:
                d_st[r - (NB - LOOKAHEAD)].wait()
            d_g[r + LOOKAHEAD] = issue_g(r + LOOKAHEAD)
        d_g[r].wait()
        if b == 0:
            d_pe[c].wait()
        add_pe(r)
        d_st[r] = issue_st(r)
        if b == BATCH - 1 and c + 2 < N_CHUNKS:
            d_pe[c + 2] = issue_pe(c + 2)
    for r in range(ROUNDS - NB, ROUNDS):
        if r in d_st:
            d_st[r].wait()


@jax.jit
def kernel(x, tok_table):
    pe = _pe_table()
    x_flat = x.reshape(BATCH * SEQ).astype(jnp.int32)

    mesh = plsc.VectorSubcoreMesh(core_axis_name="c", subcore_axis_name="s")
    run = pl.kernel(
        _sc_body,
        out_type=jax.ShapeDtypeStruct((BATCH * SEQ, D), jnp.float32),
        mesh=mesh,
        scratch_types=[
            pltpu.VMEM((BATCH, POS_PER_W), jnp.int32),
            [pltpu.VMEM((CHUNK, D), jnp.float32) for _ in range(NB)],
            [pltpu.VMEM((CHUNK, D), jnp.float32) for _ in range(NPE)],
            [pltpu.SemaphoreType.DMA for _ in range(NPE)],
            [pltpu.SemaphoreType.DMA for _ in range(NB)],
            [pltpu.SemaphoreType.DMA for _ in range(NB)],
        ],
    )
    out = run(x_flat, pe, tok_table)
    return out.reshape(BATCH, SEQ, D)


# speedup vs baseline: 1.6426x; 1.0268x over previous
import jax
import jax.numpy as jnp
import numpy as np
from jax import lax
from jax.experimental import pallas as pl
from jax.experimental.pallas import tpu as pltpu
from jax.experimental.pallas import tpu_sc as plsc

VOCAB = 100000
D = 768
BATCH = 4
SEQ = 4096
LANES = 16
D_VECS = D // LANES

NC = 2
NS = 16
NW = NC * NS

POS_PER_W = SEQ // NW
CHUNK = 32
N_CHUNKS = POS_PER_W // CHUNK
ROUNDS = N_CHUNKS * BATCH
NB = 3
NPE = 2
LOOKAHEAD = 1


def _pe_table():
    pos = np.arange(SEQ, dtype=np.float32)[:, None]
    i = np.arange(0, D, 2, dtype=np.float32)
    div = np.power(np.float32(10000.0), i / np.float32(D))
    pe = np.zeros((SEQ, D), dtype=np.float32)
    pe[:, 0::2] = np.sin(pos / div, dtype=np.float32)
    pe[:, 1::2] = np.cos(pos / div, dtype=np.float32)
    return jnp.asarray(pe)


def _sc_body(x_hbm, pe_hbm, tab_hbm, out_hbm, idx_v, rows, pe_v,
             pe_sem, g_sem, st_sem):
    wid = lax.axis_index("s") * NC + lax.axis_index("c")
    pos0 = wid * POS_PER_W

    for b in range(BATCH):
        pltpu.sync_copy(x_hbm.at[b, pl.ds(pos0, POS_PER_W)], idx_v.at[b])

    def cb(r):
        return r // BATCH, r % BATCH

    def issue_pe(c):
        return pltpu.async_copy(
            pe_hbm.at[pl.ds(pos0 + c * CHUNK, CHUNK)], pe_v[c % NPE],
            pe_sem[c % NPE])

    def issue_g(r):
        c, b = cb(r)
        return pltpu.async_copy(
            tab_hbm.at[idx_v.at[b, pl.ds(c * CHUNK, CHUNK)]],
            rows[r % NB], g_sem[r % NB])

    def issue_st(r):
        c, b = cb(r)
        dst = b * SEQ + pos0 + c * CHUNK
        return pltpu.async_copy(
            rows[r % NB], out_hbm.at[pl.ds(dst, CHUNK)], st_sem[r % NB])

    def add_pe(r):
        c, _ = cb(r)
        rbuf, pbuf = rows[r % NB], pe_v[c % NPE]

        def body(i2, _):
            for di in range(2):
                i = i2 * 2 + di
                for j in range(D_VECS):
                    sl = pl.ds(j * LANES, LANES)
                    plsc.addupdate(rbuf.at[i, sl], pbuf[i, sl])
            return 0

        lax.fori_loop(0, CHUNK // 2, body, 0)

    d_pe, d_g, d_st = {}, {}, {}
    d_pe[0] = issue_pe(0)
    d_pe[1] = issue_pe(1)
    for r in range(LOOKAHEAD):
        d_g[r] = issue_g(r)
    for r in range(ROUNDS):
        c, b = cb(r)
        if r + LOOKAHEAD < ROUNDS:
            if r - (NB - LOOKAHEAD) >= 0:
                d_st[r - (NB - LOOKAHEAD)].wait()
            d_g[r + LOOKAHEAD] = issue_g(r + LOOKAHEAD)
        d_g[r].wait()
        if b == 0:
            d_pe[c].wait()
        add_pe(r)
        d_st[r] = issue_st(r)
        if b == BATCH - 1 and c + 2 < N_CHUNKS:
            d_pe[c + 2] = issue_pe(c + 2)
    for r in range(ROUNDS - NB, ROUNDS):
        if r in d_st:
            d_st[r].wait()


@jax.jit
def kernel(x, tok_table):
    pe = _pe_table()
    x_i32 = x.astype(jnp.int32)

    mesh = plsc.VectorSubcoreMesh(core_axis_name="c", subcore_axis_name="s")
    run = pl.kernel(
        _sc_body,
        out_type=jax.ShapeDtypeStruct((BATCH * SEQ, D), jnp.float32),
        mesh=mesh,
        scratch_types=[
            pltpu.VMEM((BATCH, POS_PER_W), jnp.int32),
            [pltpu.VMEM((CHUNK, D), jnp.float32) for _ in range(NB)],
            [pltpu.VMEM((CHUNK, D), jnp.float32) for _ in range(NPE)],
            [pltpu.SemaphoreType.DMA for _ in range(NPE)],
            [pltpu.SemaphoreType.DMA for _ in range(NB)],
            [pltpu.SemaphoreType.DMA for _ in range(NB)],
        ],
    )
    out = run(x_i32, pe, tok_table)
    return out.reshape(BATCH, SEQ, D)
